# baseline (speedup 1.0000x reference)
import functools

import jax
import jax.numpy as jnp
from jax import lax
from jax.experimental import pallas as pl
from jax.experimental.pallas import tpu as pltpu
from jax.experimental.pallas import tpu_sc as plsc

PAD = 1
B, S, D = 4, 8192, 1024
N = B * S
NC, NS = 2, 16
NW = NC * NS
CHUNK = N // NW
CPR = S // CHUNK
GROWS = 32
NG = CHUNK // GROWS
LANES = 16
TABLE_ROWS = 8194


def _mask(v):
    return jnp.minimum(jnp.abs(v - PAD), 1)


def _sc_body(inp_hbm, table_hbm, out_hbm, row_v, idx_v,
             buf0, buf1, buf2, gsem0, gsem1, gsem2, osem0, osem1, osem2):
    cid = lax.axis_index("c")
    sid = lax.axis_index("s")
    wid = cid * NS + sid
    row = wid // CPR
    off = (wid % CPR) * CHUNK
    base = wid * CHUNK

    pltpu.sync_copy(inp_hbm.at[pl.ds(row * S, S)], row_v)

    def count_body(k, acc):
        a = acc
        for j in range(4):
            a = a + _mask(row_v[pl.ds(k * (4 * LANES) + j * LANES, LANES)])
        return a

    accv = lax.fori_loop(0, off // (4 * LANES), count_body,
                         jnp.zeros((LANES,), jnp.int32))
    prefix = jnp.sum(accv)

    bufs = (buf0, buf1, buf2)
    gsems = (gsem0, gsem1, gsem2)
    osems = (osem0, osem1, osem2)
    gh = [None, None, None]
    oh = [None, None, None]

    c = prefix
    for r in range(NG):
        for k in range(GROWS // LANES):
            v = row_v[pl.ds(off + r * GROWS + k * LANES, LANES)]
            m = _mask(v)
            cs = plsc.cumsum(m)
            pos = (cs + c) * m + PAD
            pos = jnp.minimum(jnp.maximum(pos, 0), TABLE_ROWS - 1)
            idx_v[r, pl.ds(k * LANES, LANES)] = pos
            c = c + jnp.sum(m)
        if r <= 1:
            gh[r] = pltpu.async_copy(
                table_hbm.at[idx_v.at[r]], bufs[r], gsems[r])

    for t in range(NG):
        b = t % 3
        gh[b].wait()
        oh[b] = pltpu.async_copy(
            bufs[b], out_hbm.at[pl.ds(base + t * GROWS, GROWS)], osems[b])
        if t + 2 < NG:
            nb = (t + 2) % 3
            if oh[nb] is not None:
                oh[nb].wait()
            gh[nb] = pltpu.async_copy(
                table_hbm.at[idx_v.at[t + 2]], bufs[nb], gsems[nb])
    oh[(NG - 3) % 3].wait()
    oh[(NG - 2) % 3].wait()
    oh[(NG - 1) % 3].wait()


_sc_call = functools.partial(
    pl.kernel,
    out_type=jax.ShapeDtypeStruct((N, D), jnp.float32),
    mesh=plsc.VectorSubcoreMesh(core_axis_name="c", subcore_axis_name="s"),
    scratch_types=[
        pltpu.VMEM((S,), jnp.int32),
        pltpu.VMEM((NG, GROWS), jnp.int32),
        pltpu.VMEM((GROWS, D), jnp.float32),
        pltpu.VMEM((GROWS, D), jnp.float32),
        pltpu.VMEM((GROWS, D), jnp.float32),
        pltpu.SemaphoreType.DMA,
        pltpu.SemaphoreType.DMA,
        pltpu.SemaphoreType.DMA,
        pltpu.SemaphoreType.DMA,
        pltpu.SemaphoreType.DMA,
        pltpu.SemaphoreType.DMA,
    ],
    compiler_params=pltpu.CompilerParams(needs_layout_passes=False),
)(_sc_body)


def kernel(input, table):
    inp = input.reshape(N).astype(jnp.int32)
    out = _sc_call(inp, table)
    return out.reshape(B, S, D)


# speedup vs baseline: 1.0111x; 1.0111x over previous
import functools

import jax
import jax.numpy as jnp
from jax import lax
from jax.experimental import pallas as pl
from jax.experimental.pallas import tpu as pltpu
from jax.experimental.pallas import tpu_sc as plsc

PAD = 1
B, S, D = 4, 8192, 1024
N = B * S
NC, NS = 2, 16
NW = NC * NS
CHUNK = N // NW
CPR = S // CHUNK
GROWS = 32
NG = CHUNK // GROWS
LANES = 16
TABLE_ROWS = 8194


def _mask(v):
    return jnp.minimum(jnp.abs(v - PAD), 1)


def _sc_body(inp_hbm, table_hbm, out_hbm, row_v, idx_v,
             buf0, buf1, buf2, gsem0, gsem1, gsem2, osem0, osem1, osem2):
    cid = lax.axis_index("c")
    sid = lax.axis_index("s")
    wid = cid * NS + sid
    row = wid // CPR
    off = (wid % CPR) * CHUNK
    base = wid * CHUNK

    pltpu.sync_copy(inp_hbm.at[pl.ds(row * S, S)], row_v)

    def count_body(k, acc):
        a = acc
        for j in range(4):
            a = a + _mask(row_v[pl.ds(k * (4 * LANES) + j * LANES, LANES)])
        return a

    accv = lax.fori_loop(0, off // (4 * LANES), count_body,
                         jnp.zeros((LANES,), jnp.int32))
    prefix = jnp.sum(accv)

    def pos_body(r, carry):
        c = carry
        for k in range(GROWS // LANES):
            v = row_v[pl.ds(off + r * GROWS + k * LANES, LANES)]
            m = _mask(v)
            cs = plsc.cumsum(m)
            pos = (cs + c) * m + PAD
            pos = jnp.minimum(jnp.maximum(pos, 0), TABLE_ROWS - 1)
            idx_v[r, pl.ds(k * LANES, LANES)] = pos
            c = c + jnp.sum(m)
        return c

    lax.fori_loop(0, NG, pos_body, prefix)

    bufs = (buf0, buf1, buf2)
    gsems = (gsem0, gsem1, gsem2)
    osems = (osem0, osem1, osem2)
    gh = [None, None, None]
    oh = [None, None, None]
    gh[0] = pltpu.async_copy(table_hbm.at[idx_v.at[0]], bufs[0], gsems[0])
    gh[1] = pltpu.async_copy(table_hbm.at[idx_v.at[1]], bufs[1], gsems[1])
    for t in range(NG):
        b = t % 3
        gh[b].wait()
        oh[b] = pltpu.async_copy(
            bufs[b], out_hbm.at[pl.ds(base + t * GROWS, GROWS)], osems[b])
        if t + 2 < NG:
            nb = (t + 2) % 3
            if oh[nb] is not None:
                oh[nb].wait()
            gh[nb] = pltpu.async_copy(
                table_hbm.at[idx_v.at[t + 2]], bufs[nb], gsems[nb])
    oh[(NG - 3) % 3].wait()
    oh[(NG - 2) % 3].wait()
    oh[(NG - 1) % 3].wait()


_sc_call = functools.partial(
    pl.kernel,
    out_type=jax.ShapeDtypeStruct((N, D), jnp.float32),
    mesh=plsc.VectorSubcoreMesh(core_axis_name="c", subcore_axis_name="s"),
    scratch_types=[
        pltpu.VMEM((S,), jnp.int32),
        pltpu.VMEM((NG, GROWS), jnp.int32),
        pltpu.VMEM((GROWS, D), jnp.float32),
        pltpu.VMEM((GROWS, D), jnp.float32),
        pltpu.VMEM((GROWS, D), jnp.float32),
        pltpu.SemaphoreType.DMA,
        pltpu.SemaphoreType.DMA,
        pltpu.SemaphoreType.DMA,
        pltpu.SemaphoreType.DMA,
        pltpu.SemaphoreType.DMA,
        pltpu.SemaphoreType.DMA,
    ],
    compiler_params=pltpu.CompilerParams(needs_layout_passes=False),
)(_sc_body)


def kernel(input, table):
    inp = input.reshape(N).astype(jnp.int32)
    out = _sc_call(inp, table)
    return out.reshape(B, S, D)


# speedup vs baseline: 1.0127x; 1.0016x over previous
import functools

import jax
import jax.numpy as jnp
from jax import lax
from jax.experimental import pallas as pl
from jax.experimental.pallas import tpu as pltpu
from jax.experimental.pallas import tpu_sc as plsc

PAD = 1
B, S, D = 4, 8192, 1024
N = B * S
NC, NS = 2, 16
NW = NC * NS
CHUNK = N // NW
CPR = S // CHUNK
GROWS = 32
NG = CHUNK // GROWS
LANES = 16
TABLE_ROWS = 8194


def _mask(v):
    return jnp.minimum(jnp.abs(v - PAD), 1)


def _sc_body(inp_hbm, table_hbm, out_hbm, row_v, idx_v,
             buf0, buf1, buf2, gsem0, gsem1, gsem2, osem0, osem1, osem2):
    cid = lax.axis_index("c")
    sid = lax.axis_index("s")
    wid = cid * NS + sid
    row = wid // CPR
    off = (wid % CPR) * CHUNK
    base = wid * CHUNK

    pltpu.sync_copy(inp_hbm.at[pl.ds(row * S, S)], row_v)

    def count_body(k, acc):
        a = acc
        for j in range(4):
            a = a + _mask(row_v[pl.ds(k * (4 * LANES) + j * LANES, LANES)])
        return a

    accv = lax.fori_loop(0, off // (4 * LANES), count_body,
                         jnp.zeros((LANES,), jnp.int32))
    prefix = jnp.sum(accv)

    def pos_body(r, carry):
        c = carry
        for k in range(GROWS // LANES):
            v = row_v[pl.ds(off + r * GROWS + k * LANES, LANES)]
            m = _mask(v)
            cs = plsc.cumsum(m)
            pos = (cs + c) * m + PAD
            pos = jnp.minimum(jnp.maximum(pos, 0), TABLE_ROWS - 1)
            idx_v[r, pl.ds(k * LANES, LANES)] = pos
            c = c + jnp.sum(m)
        return c

    lax.fori_loop(0, NG, pos_body, prefix)

    bufs = (buf0, buf1, buf2)
    gsems = (gsem0, gsem1, gsem2)
    osems = (osem0, osem1, osem2)
    gh = [None, None, None]
    oh = [None, None, None]
    gh[0] = pltpu.async_copy(table_hbm.at[idx_v.at[0]], bufs[0], gsems[0])
    gh[1] = pltpu.async_copy(table_hbm.at[idx_v.at[1]], bufs[1], gsems[1])
    for t in range(NG):
        b = t % 3
        gh[b].wait()
        if t + 2 < NG:
            nb = (t + 2) % 3
            if oh[nb] is not None:
                oh[nb].wait()
            gh[nb] = pltpu.async_copy(
                table_hbm.at[idx_v.at[t + 2]], bufs[nb], gsems[nb])
        oh[b] = pltpu.async_copy(
            bufs[b], out_hbm.at[pl.ds(base + t * GROWS, GROWS)], osems[b])
    oh[(NG - 3) % 3].wait()
    oh[(NG - 2) % 3].wait()
    oh[(NG - 1) % 3].wait()


_sc_call = functools.partial(
    pl.kernel,
    out_type=jax.ShapeDtypeStruct((N, D), jnp.float32),
    mesh=plsc.VectorSubcoreMesh(core_axis_name="c", subcore_axis_name="s"),
    scratch_types=[
        pltpu.VMEM((S,), jnp.int32),
        pltpu.VMEM((NG, GROWS), jnp.int32),
        pltpu.VMEM((GROWS, D), jnp.float32),
        pltpu.VMEM((GROWS, D), jnp.float32),
        pltpu.VMEM((GROWS, D), jnp.float32),
        pltpu.SemaphoreType.DMA,
        pltpu.SemaphoreType.DMA,
        pltpu.SemaphoreType.DMA,
        pltpu.SemaphoreType.DMA,
        pltpu.SemaphoreType.DMA,
        pltpu.SemaphoreType.DMA,
    ],
    compiler_params=pltpu.CompilerParams(needs_layout_passes=False),
)(_sc_body)


def kernel(input, table):
    inp = input.reshape(N).astype(jnp.int32)
    out = _sc_call(inp, table)
    return out.reshape(B, S, D)
